# baseline (speedup 1.0000x reference)
import functools

import jax
import jax.numpy as jnp
from jax import lax
from jax.experimental import pallas as pl
from jax.experimental.pallas import tpu as pltpu
from jax.experimental.pallas import tpu_sc as plsc

_NC = 2
_NS = 16
_NW = _NC * _NS
_L = 16

_NSEG = 8
_C = 32
_NBUF = 4


@functools.lru_cache(maxsize=None)
def _make_kernel(T, D):
    assert T % _NW == 0
    rows_per_w = T // _NW
    n_chunks = rows_per_w // _C
    assert rows_per_w % _C == 0

    mesh = plsc.VectorSubcoreMesh(core_axis_name="c", subcore_axis_name="s",
                                  num_cores=_NC, num_subcores=_NS)

    @functools.partial(
        pl.kernel,
        out_type=(
            jax.ShapeDtypeStruct((2 * T * D,), jnp.float32),
            jax.ShapeDtypeStruct((_L,), jnp.int32),
        ),
        mesh=mesh,
        compiler_params=pltpu.CompilerParams(needs_layout_passes=False),
        scratch_types=[
            pltpu.VMEM((_L,), jnp.int32),
            pltpu.VMEM((_L,), jnp.int32),
            pltpu.VMEM((_L,), jnp.int32),
            [pltpu.SemaphoreType.DMA] * _NBUF,
        ],
    )
    def k(flat1_hbm, cu1_hbm, flat2_hbm, cu2_hbm, out_hbm, cuout_hbm,
          cu1_v, cu2_v, cuout_v, semS):
        wid = lax.axis_index("s") * _NC + lax.axis_index("c")
        iota = lax.iota(jnp.int32, _L)

        base_w = wid * rows_per_w
        chunks = ([(flat1_hbm, base_w + c * _C, 0) for c in range(n_chunks)]
                  + [(flat2_hbm, base_w + c * _C, 1) for c in range(n_chunks)])
        nch = len(chunks)

        pltpu.sync_copy(cu1_hbm, cu1_v)
        pltpu.sync_copy(cu2_hbm, cu2_v)
        cu1 = cu1_v[...]
        cu2 = cu2_v[...]

        s_cu1 = [jnp.sum(jnp.where(iota == b, cu1, 0)) for b in range(_NSEG + 1)]
        s_cu2 = [jnp.sum(jnp.where(iota == b, cu2, 0)) for b in range(_NSEG + 1)]
        len1 = [s_cu1[b + 1] - s_cu1[b] for b in range(_NSEG)]
        len2 = [s_cu2[b + 1] - s_cu2[b] for b in range(_NSEG)]
        cu_out = [jnp.int32(0)]
        for b in range(_NSEG):
            cu_out.append(cu_out[b] + len1[b] + len2[b])
        off1 = [cu_out[b] - s_cu1[b] for b in range(_NSEG + 1)]
        off2 = [cu_out[b] + (len1[b] if b < _NSEG else 0) - s_cu2[b]
                for b in range(_NSEG + 1)]

        cuvec = jnp.zeros((_L,), jnp.int32)
        for b in range(_NSEG + 1):
            cuvec = jnp.where(iota == b, cu_out[b], cuvec)

        @pl.when(wid == 0)
        def _():
            cuout_v[...] = cuvec
            pltpu.sync_copy(cuout_v, cuout_hbm)

        def dest_scalar(t, s_cu, off):
            d = t + off[0]
            for b in range(1, _NSEG + 1):
                d = d + jnp.where(t >= s_cu[b], off[b] - off[b - 1], 0)
            return d

        def drain(i):
            pltpu.make_async_copy(flat1_hbm.at[pl.ds(0, _C * D)],
                                  out_hbm.at[pl.ds(0, _C * D)],
                                  semS[i]).wait()

        for c in range(nch):
            i = c % _NBUF
            ref, base, which = chunks[c]
            s_cu, off = (s_cu1, off1) if which == 0 else (s_cu2, off2)
            if c >= _NBUF:
                drain(i)
            d_first = dest_scalar(base, s_cu, off)
            d_last = dest_scalar(base + (_C - 1), s_cu, off)
            contiguous = (d_last - d_first) == (_C - 1)

            @pl.when(contiguous)
            def _():
                pltpu.async_copy(ref.at[pl.ds(base * D, _C * D)],
                                 out_hbm.at[pl.ds(d_first * D, _C * D)],
                                 semS[i])

            @pl.when(jnp.logical_not(contiguous))
            def _():
                def body(r, carry):
                    t = base + r
                    dr = dest_scalar(t, s_cu, off)
                    pltpu.async_copy(ref.at[pl.ds(t * D, D)],
                                     out_hbm.at[pl.ds(dr * D, D)],
                                     semS[i])
                    return carry
                lax.fori_loop(0, _C, body, jnp.int32(0))

        for c in range(max(0, nch - _NBUF), nch):
            drain(c % _NBUF)

    return k


def kernel(flat1, cu_seqlens1, flat2, cu_seqlens2):
    T, D = flat1.shape
    k = _make_kernel(T, D)
    pad = jnp.full((_L - cu_seqlens1.shape[0],), T, jnp.int32)
    cu1p = jnp.concatenate([cu_seqlens1.astype(jnp.int32), pad])
    cu2p = jnp.concatenate([cu_seqlens2.astype(jnp.int32), pad])
    out_flat, cu_out_pad = k(flat1.reshape(-1), cu1p, flat2.reshape(-1), cu2p)
    return out_flat.reshape(2 * T, D), cu_out_pad[: cu_seqlens1.shape[0]]


# speedup vs baseline: 11.2475x; 11.2475x over previous
import functools

import jax
import jax.numpy as jnp
from jax import lax
from jax.experimental import pallas as pl
from jax.experimental.pallas import tpu as pltpu
from jax.experimental.pallas import tpu_sc as plsc

_NC = 2
_NS = 16
_NW = _NC * _NS
_L = 16

_NSEG = 8
_C = 32
_NBUF = 3


@functools.lru_cache(maxsize=None)
def _make_kernel(T, D):
    assert T % _NW == 0
    rows_per_w = T // _NW
    n_chunks = rows_per_w // _C
    assert rows_per_w % _C == 0

    mesh = plsc.VectorSubcoreMesh(core_axis_name="c", subcore_axis_name="s",
                                  num_cores=_NC, num_subcores=_NS)

    @functools.partial(
        pl.kernel,
        out_type=(
            jax.ShapeDtypeStruct((2 * T * D,), jnp.float32),
            jax.ShapeDtypeStruct((_L,), jnp.int32),
        ),
        mesh=mesh,
        compiler_params=pltpu.CompilerParams(needs_layout_passes=False),
        scratch_types=[
            pltpu.VMEM((_L,), jnp.int32),
            pltpu.VMEM((_L,), jnp.int32),
            pltpu.VMEM((_L,), jnp.int32),
            [pltpu.VMEM((_C * D,), jnp.float32)] * _NBUF,
            [pltpu.SemaphoreType.DMA] * _NBUF,
            [pltpu.SemaphoreType.DMA] * _NBUF,
        ],
    )
    def k(flat1_hbm, cu1_hbm, flat2_hbm, cu2_hbm, out_hbm, cuout_hbm,
          cu1_v, cu2_v, cuout_v, dbufs, semL, semS):
        wid = lax.axis_index("s") * _NC + lax.axis_index("c")
        iota = lax.iota(jnp.int32, _L)

        base_w = wid * rows_per_w
        chunks = ([(flat1_hbm, base_w + c * _C, 0) for c in range(n_chunks)]
                  + [(flat2_hbm, base_w + c * _C, 1) for c in range(n_chunks)])
        nch = len(chunks)

        def start_load(c):
            ref, base, _ = chunks[c]
            i = c % _NBUF
            return pltpu.async_copy(ref.at[pl.ds(base * D, _C * D)],
                                    dbufs[i], semL[i])

        loadd = {0: start_load(0), 1: start_load(1)}

        pltpu.sync_copy(cu1_hbm, cu1_v)
        pltpu.sync_copy(cu2_hbm, cu2_v)
        cu1 = cu1_v[...]
        cu2 = cu2_v[...]

        s_cu1 = [jnp.sum(jnp.where(iota == b, cu1, 0)) for b in range(_NSEG + 1)]
        s_cu2 = [jnp.sum(jnp.where(iota == b, cu2, 0)) for b in range(_NSEG + 1)]
        len1 = [s_cu1[b + 1] - s_cu1[b] for b in range(_NSEG)]
        len2 = [s_cu2[b + 1] - s_cu2[b] for b in range(_NSEG)]
        cu_out = [jnp.int32(0)]
        for b in range(_NSEG):
            cu_out.append(cu_out[b] + len1[b] + len2[b])
        off1 = [cu_out[b] - s_cu1[b] for b in range(_NSEG + 1)]
        off2 = [cu_out[b] + (len1[b] if b < _NSEG else 0) - s_cu2[b]
                for b in range(_NSEG + 1)]

        cuvec = jnp.zeros((_L,), jnp.int32)
        for b in range(_NSEG + 1):
            cuvec = jnp.where(iota == b, cu_out[b], cuvec)

        @pl.when(wid == 0)
        def _():
            cuout_v[...] = cuvec
            pltpu.sync_copy(cuout_v, cuout_hbm)

        def dest_scalar(t, s_cu, off):
            d = t + off[0]
            for b in range(1, _NSEG + 1):
                d = d + jnp.where(t >= s_cu[b], off[b] - off[b - 1], 0)
            return d

        def drain_store(i):
            pltpu.make_async_copy(dbufs[i], out_hbm.at[pl.ds(0, _C * D)],
                                  semS[i]).wait()

        for c in range(nch):
            i = c % _NBUF
            _, base, which = chunks[c]
            s_cu, off = (s_cu1, off1) if which == 0 else (s_cu2, off2)
            loadd[c].wait()
            d_first = dest_scalar(base, s_cu, off)
            d_last = dest_scalar(base + (_C - 1), s_cu, off)
            contiguous = (d_last - d_first) == (_C - 1)

            @pl.when(contiguous)
            def _():
                pltpu.async_copy(dbufs[i],
                                 out_hbm.at[pl.ds(d_first * D, _C * D)],
                                 semS[i])

            @pl.when(jnp.logical_not(contiguous))
            def _():
                def body(r, carry):
                    dr = dest_scalar(base + r, s_cu, off)
                    pltpu.async_copy(dbufs[i].at[pl.ds(r * D, D)],
                                     out_hbm.at[pl.ds(dr * D, D)],
                                     semS[i])
                    return carry
                lax.fori_loop(0, _C, body, jnp.int32(0))

            if c + 2 < nch:
                if c >= 1:
                    drain_store((c - 1) % _NBUF)
                loadd[c + 2] = start_load(c + 2)
        for c in range(nch - _NBUF, nch):
            drain_store(c % _NBUF)

    return k


def kernel(flat1, cu_seqlens1, flat2, cu_seqlens2):
    T, D = flat1.shape
    k = _make_kernel(T, D)
    pad = jnp.full((_L - cu_seqlens1.shape[0],), T, jnp.int32)
    cu1p = jnp.concatenate([cu_seqlens1.astype(jnp.int32), pad])
    cu2p = jnp.concatenate([cu_seqlens2.astype(jnp.int32), pad])
    out_flat, cu_out_pad = k(flat1.reshape(-1), cu1p, flat2.reshape(-1), cu2p)
    return out_flat.reshape(2 * T, D), cu_out_pad[: cu_seqlens1.shape[0]]


# speedup vs baseline: 27.0819x; 2.4078x over previous
import functools

import jax
import jax.numpy as jnp
from jax import lax
from jax.experimental import pallas as pl
from jax.experimental.pallas import tpu as pltpu
from jax.experimental.pallas import tpu_sc as plsc

_NC = 2
_NS = 16
_NW = _NC * _NS
_L = 16

_NSEG = 8
_NRUN = 2 * _NSEG
_C = 32
_NBUF = 3


@functools.lru_cache(maxsize=None)
def _make_kernel(T, D):
    out_rows = 2 * T
    assert out_rows % (_NW * _C) == 0
    rows_per_w = out_rows // _NW
    n_chunks = rows_per_w // _C
    src_per_w = T // _NW
    n_groups = src_per_w // _L

    mesh = plsc.VectorSubcoreMesh(core_axis_name="c", subcore_axis_name="s",
                                  num_cores=_NC, num_subcores=_NS)

    @functools.partial(
        pl.kernel,
        out_type=(
            jax.ShapeDtypeStruct((out_rows, D), jnp.float32),
            jax.ShapeDtypeStruct((_L,), jnp.int32),
        ),
        mesh=mesh,
        compiler_params=pltpu.CompilerParams(needs_layout_passes=False),
        scratch_types=[
            pltpu.VMEM((_L,), jnp.int32),
            pltpu.VMEM((_L,), jnp.int32),
            pltpu.VMEM((_L,), jnp.int32),
            [pltpu.VMEM((_C, D), jnp.float32)] * _NBUF,
            [pltpu.VMEM((_C,), jnp.int32)] * _NBUF,
            pltpu.VMEM((_L, D), jnp.float32),
            pltpu.VMEM((_L,), jnp.int32),
            [pltpu.SemaphoreType.DMA] * _NBUF,
            [pltpu.SemaphoreType.DMA] * _NBUF,
            pltpu.SemaphoreType.DMA,
        ],
    )
    def k(flat1_hbm, cu1_hbm, flat2_hbm, cu2_hbm, out_hbm, cuout_hbm,
          cu1_v, cu2_v, cuout_v, dbufs, ibufs, pbuf, ibufP, semL, semS, semP):
        wid = lax.axis_index("s") * _NC + lax.axis_index("c")
        iota = lax.iota(jnp.int32, _L)

        pltpu.sync_copy(cu1_hbm, cu1_v)
        pltpu.sync_copy(cu2_hbm, cu2_v)
        cu1 = cu1_v[...]
        cu2 = cu2_v[...]

        s_cu1 = [jnp.sum(jnp.where(iota == b, cu1, 0)) for b in range(_NSEG + 1)]
        s_cu2 = [jnp.sum(jnp.where(iota == b, cu2, 0)) for b in range(_NSEG + 1)]
        len1 = [s_cu1[b + 1] - s_cu1[b] for b in range(_NSEG)]
        len2 = [s_cu2[b + 1] - s_cu2[b] for b in range(_NSEG)]
        cu_out = [jnp.int32(0)]
        for b in range(_NSEG):
            cu_out.append(cu_out[b] + len1[b] + len2[b])
        fwd1 = [cu_out[b] - s_cu1[b] for b in range(_NSEG + 1)]
        fwd2 = [cu_out[b] + (len1[b] if b < _NSEG else 0) - s_cu2[b]
                for b in range(_NSEG + 1)]
        rst = []
        ioff = []
        for b in range(_NSEG):
            rst.append(cu_out[b])
            ioff.append(s_cu1[b] - cu_out[b])
            rst.append(cu_out[b] + len1[b])
            ioff.append(s_cu2[b] - cu_out[b] - len1[b])

        cuvec = jnp.zeros((_L,), jnp.int32)
        for b in range(_NSEG + 1):
            cuvec = jnp.where(iota == b, cu_out[b], cuvec)

        @pl.when(wid == 0)
        def _():
            cuout_v[...] = cuvec
            pltpu.sync_copy(cuout_v, cuout_hbm)

        def run_of(o):
            s = jnp.int32(0) if not hasattr(o, "shape") or o.shape == () \
                else jnp.zeros((_L,), jnp.int32)
            for kk in range(1, _NRUN):
                s = s + jnp.where(o >= rst[kk], 1, 0)
            return s

        def src_of(o):
            v = o + ioff[0]
            for kk in range(1, _NRUN):
                v = v + jnp.where(o >= rst[kk], ioff[kk] - ioff[kk - 1], 0)
            return v

        base_w = wid * rows_per_w

        def compute_ibuf(c):
            i = c % _NBUF
            obase = base_w + c * _C
            for j in range(_C // _L):
                o = obase + j * _L + iota
                ibufs[i][pl.ds(j * _L, _L)] = jnp.clip(src_of(o), 0, T - 1)

        def start_gather(c):
            i = c % _NBUF
            obase = base_w + c * _C
            from1 = (run_of(obase) % 2) == 0

            @pl.when(from1)
            def _():
                pltpu.async_copy(flat1_hbm.at[ibufs[i]], dbufs[i], semL[i])

            @pl.when(jnp.logical_not(from1))
            def _():
                pltpu.async_copy(flat2_hbm.at[ibufs[i]], dbufs[i], semL[i])

        def drain_gather(i):
            pltpu.make_async_copy(flat1_hbm.at[pl.ds(0, _C)], dbufs[i],
                                  semL[i]).wait()

        def drain_store(i):
            pltpu.make_async_copy(dbufs[i], out_hbm.at[pl.ds(0, _C)],
                                  semS[i]).wait()

        compute_ibuf(0)
        start_gather(0)
        if n_chunks > 1:
            compute_ibuf(1)
            start_gather(1)

        store_flag = {}
        for c in range(n_chunks):
            i = c % _NBUF
            obase = base_w + c * _C
            drain_gather(i)
            single = run_of(obase) == run_of(obase + (_C - 1))

            @pl.when(single)
            def _():
                pltpu.async_copy(dbufs[i], out_hbm.at[pl.ds(obase, _C)],
                                 semS[i])

            store_flag[c] = single
            if c + 2 < n_chunks:
                if c >= 1:
                    @pl.when(store_flag[c - 1])
                    def _():
                        drain_store((c - 1) % _NBUF)
                compute_ibuf(c + 2)
                start_gather(c + 2)
        for c in range(max(0, n_chunks - 3), n_chunks):
            @pl.when(store_flag[c])
            def _():
                drain_store(c % _NBUF)

        sbase_w = wid * src_per_w

        def phase2(src_hbm, s_cu, fwd):
            def dest_vec(t):
                d = t + fwd[0]
                for b in range(1, _NSEG + 1):
                    d = d + jnp.where(t >= s_cu[b], fwd[b] - fwd[b - 1], 0)
                return d

            def body(h, carry):
                base = pl.multiple_of(sbase_w + h * _L, _L)
                t = base + iota
                d = dest_vec(t)
                cs = lax.bitwise_and(d, jnp.int32(-_C))
                mixed = run_of(cs) != run_of(cs + (_C - 1))
                anym = jnp.max(jnp.where(mixed, 1, 0))

                @pl.when(anym == 1)
                def _():
                    pltpu.sync_copy(src_hbm.at[pl.ds(base, _L)], pbuf)
                    ibufP[...] = d
                    pltpu.async_copy(pbuf, out_hbm.at[ibufP], semP).wait()
                return carry
            lax.fori_loop(0, n_groups, body, jnp.int32(0))

        phase2(flat1_hbm, s_cu1, fwd1)
        phase2(flat2_hbm, s_cu2, fwd2)

    return k


def kernel(flat1, cu_seqlens1, flat2, cu_seqlens2):
    T, D = flat1.shape
    k = _make_kernel(T, D)
    pad = jnp.full((_L - cu_seqlens1.shape[0],), T, jnp.int32)
    cu1p = jnp.concatenate([cu_seqlens1.astype(jnp.int32), pad])
    cu2p = jnp.concatenate([cu_seqlens2.astype(jnp.int32), pad])
    out, cu_out_pad = k(flat1, cu1p, flat2, cu2p)
    return out, cu_out_pad[: cu_seqlens1.shape[0]]


# speedup vs baseline: 31.4408x; 1.1610x over previous
import functools

import jax
import jax.numpy as jnp
from jax import lax
from jax.experimental import pallas as pl
from jax.experimental.pallas import tpu as pltpu
from jax.experimental.pallas import tpu_sc as plsc

_NC = 2
_NS = 16
_NW = _NC * _NS
_L = 16

_NSEG = 8
_C = 32
_NBUF = 3


@functools.lru_cache(maxsize=None)
def _make_kernel(T, D):
    assert T % _NW == 0
    rows_per_w = T // _NW
    n_chunks = rows_per_w // _C
    assert rows_per_w % _C == 0

    mesh = plsc.VectorSubcoreMesh(core_axis_name="c", subcore_axis_name="s",
                                  num_cores=_NC, num_subcores=_NS)

    @functools.partial(
        pl.kernel,
        out_type=(
            jax.ShapeDtypeStruct((2 * T, D), jnp.float32),
            jax.ShapeDtypeStruct((_L,), jnp.int32),
        ),
        mesh=mesh,
        compiler_params=pltpu.CompilerParams(needs_layout_passes=False),
        scratch_types=[
            pltpu.VMEM((_L,), jnp.int32),
            pltpu.VMEM((_L,), jnp.int32),
            pltpu.VMEM((_L,), jnp.int32),
            [pltpu.VMEM((_C, D), jnp.float32)] * _NBUF,
            [pltpu.VMEM((_C,), jnp.int32)] * _NBUF,
            [pltpu.SemaphoreType.DMA] * _NBUF,
            [pltpu.SemaphoreType.DMA] * _NBUF,
        ],
    )
    def k(flat1_hbm, cu1_hbm, flat2_hbm, cu2_hbm, out_hbm, cuout_hbm,
          cu1_v, cu2_v, cuout_v, dbufs, ibufs, semL, semS):
        wid = lax.axis_index("s") * _NC + lax.axis_index("c")
        iota = lax.iota(jnp.int32, _L)

        base_w = wid * rows_per_w
        chunks = ([(flat1_hbm, base_w + c * _C, 0) for c in range(n_chunks)]
                  + [(flat2_hbm, base_w + c * _C, 1) for c in range(n_chunks)])
        nch = len(chunks)

        def start_load(c):
            ref, base, _ = chunks[c]
            i = c % _NBUF
            return pltpu.async_copy(ref.at[pl.ds(base, _C)], dbufs[i], semL[i])

        loadd = {0: start_load(0), 1: start_load(1)}

        pltpu.sync_copy(cu1_hbm, cu1_v)
        pltpu.sync_copy(cu2_hbm, cu2_v)
        cu1 = cu1_v[...]
        cu2 = cu2_v[...]

        s_cu1 = [jnp.sum(jnp.where(iota == b, cu1, 0)) for b in range(_NSEG + 1)]
        s_cu2 = [jnp.sum(jnp.where(iota == b, cu2, 0)) for b in range(_NSEG + 1)]
        len1 = [s_cu1[b + 1] - s_cu1[b] for b in range(_NSEG)]
        len2 = [s_cu2[b + 1] - s_cu2[b] for b in range(_NSEG)]
        cu_out = [jnp.int32(0)]
        for b in range(_NSEG):
            cu_out.append(cu_out[b] + len1[b] + len2[b])
        off1 = [cu_out[b] - s_cu1[b] for b in range(_NSEG + 1)]
        off2 = [cu_out[b] + (len1[b] if b < _NSEG else 0) - s_cu2[b]
                for b in range(_NSEG + 1)]

        cuvec = jnp.zeros((_L,), jnp.int32)
        for b in range(_NSEG + 1):
            cuvec = jnp.where(iota == b, cu_out[b], cuvec)

        @pl.when(wid == 0)
        def _():
            cuout_v[...] = cuvec
            pltpu.sync_copy(cuout_v, cuout_hbm)

        scatd = {}
        waited = set()
        for c in range(nch):
            i = c % _NBUF
            loadd[c].wait()
            _, base, which = chunks[c]
            s_cu, off = (s_cu1, off1) if which == 0 else (s_cu2, off2)
            for j in range(_C // _L):
                t = base + j * _L + iota
                d = t + off[0]
                for b in range(1, _NSEG + 1):
                    d = d + jnp.where(t >= s_cu[b], off[b] - off[b - 1], 0)
                ibufs[i][pl.ds(j * _L, _L)] = d
            scatd[c] = pltpu.async_copy(dbufs[i], out_hbm.at[ibufs[i]], semS[i])
            if c + 2 < nch:
                if c >= 1:
                    scatd[c - 1].wait()
                    waited.add(c - 1)
                loadd[c + 2] = start_load(c + 2)
        for c in range(nch):
            if c not in waited:
                scatd[c].wait()

    return k


def kernel(flat1, cu_seqlens1, flat2, cu_seqlens2):
    T, D = flat1.shape
    k = _make_kernel(T, D)
    pad = jnp.full((_L - cu_seqlens1.shape[0],), T, jnp.int32)
    cu1p = jnp.concatenate([cu_seqlens1.astype(jnp.int32), pad])
    cu2p = jnp.concatenate([cu_seqlens2.astype(jnp.int32), pad])
    out, cu_out_pad = k(flat1, cu1p, flat2, cu2p)
    return out, cu_out_pad[: cu_seqlens1.shape[0]]
